# baseline (speedup 1.0000x reference)
import jax
import jax.numpy as jnp
from jax.experimental import pallas as pl
from jax.experimental.pallas import tpu as pltpu


def _body(Whr_ref, Wwr_ref, out_ref, ehlr_ref, ew2_ref):
    b = pl.program_id(0)
    _, H, _, WD = out_ref.shape
    W = ew2_ref.shape[0]
    D = WD // W

    @pl.when(b == 0)
    def _init():
        for k in range(W):
            ehlr_ref[:, pl.ds(k * D, D)] = Whr_ref[...]
        for j in range(W):
            for k in range(W):
                ew2_ref[j : j + 1, pl.ds(k * D, D)] = Wwr_ref[
                    W - 1 - j + k : W - j + k, :
                ]

    ew2 = ew2_ref[...]
    for i in range(H):
        out_ref[0, i] = ehlr_ref[H - 1 - i : 2 * H - 1 - i, :] + ew2


def kernel(x, Wh, Ww):
    B, C, H, W = x.shape
    D = Wh.shape[1]
    Whr = jnp.concatenate([Wh[::-1], jnp.zeros((1, D), Wh.dtype)], axis=0)
    Wwr = jnp.concatenate([Ww[::-1], jnp.zeros((1, D), Ww.dtype)], axis=0)
    out = pl.pallas_call(
        _body,
        grid=(B,),
        in_specs=[
            pl.BlockSpec((2 * H, D), lambda b: (0, 0)),
            pl.BlockSpec((2 * W, D), lambda b: (0, 0)),
        ],
        out_specs=pl.BlockSpec((1, H, H, W * D), lambda b: (b, 0, 0, 0)),
        out_shape=jax.ShapeDtypeStruct((B, H, H, W * D), jnp.float32),
        scratch_shapes=[
            pltpu.VMEM((2 * H, W * D), jnp.float32),
            pltpu.VMEM((W, W * D), jnp.float32),
        ],
    )(Whr, Wwr)
    return out.reshape(B, H, H, W, D)


# speedup vs baseline: 1.0148x; 1.0148x over previous
import jax
import jax.numpy as jnp
from jax.experimental import pallas as pl
from jax.experimental.pallas import tpu as pltpu


def _body(Whr_ref, Wwr_ref, out_ref, ehlr_ref, ew2_ref):
    b = pl.program_id(0)
    _, H, _, WD = out_ref.shape
    W = ew2_ref.shape[0]
    D = WD // W

    @pl.when(b == 0)
    def _init():
        for k in range(W):
            ehlr_ref[:, pl.ds(k * D, D)] = Whr_ref[...]
        for j in range(W):
            for k in range(W):
                ew2_ref[j : j + 1, pl.ds(k * D, D)] = Wwr_ref[
                    W - 1 - j + k : W - j + k, :
                ]

    BJ = 8
    for jc in range(H // BJ):
        ew2c = ew2_ref[jc * BJ : (jc + 1) * BJ, :]
        for i in range(H):
            lo = H - 1 - i + jc * BJ
            out_ref[0, i, jc * BJ : (jc + 1) * BJ] = (
                ehlr_ref[lo : lo + BJ, :] + ew2c
            )


def kernel(x, Wh, Ww):
    B, C, H, W = x.shape
    D = Wh.shape[1]
    Whr = jnp.concatenate([Wh[::-1], jnp.zeros((1, D), Wh.dtype)], axis=0)
    Wwr = jnp.concatenate([Ww[::-1], jnp.zeros((1, D), Ww.dtype)], axis=0)
    out = pl.pallas_call(
        _body,
        grid=(B,),
        in_specs=[
            pl.BlockSpec((2 * H, D), lambda b: (0, 0)),
            pl.BlockSpec((2 * W, D), lambda b: (0, 0)),
        ],
        out_specs=pl.BlockSpec((1, H, H, W * D), lambda b: (b, 0, 0, 0)),
        out_shape=jax.ShapeDtypeStruct((B, H, H, W * D), jnp.float32),
        scratch_shapes=[
            pltpu.VMEM((2 * H, W * D), jnp.float32),
            pltpu.VMEM((W, W * D), jnp.float32),
        ],
    )(Whr, Wwr)
    return out.reshape(B, H, H, W, D)


# speedup vs baseline: 3.5566x; 3.5048x over previous
import jax
import jax.numpy as jnp
from jax.experimental import pallas as pl
from jax.experimental.pallas import tpu as pltpu


def _body(Whr_ref, Wwr_ref, out_ref, eh_ref, ew_ref):
    b = pl.program_id(0)
    ib = pl.program_id(1)
    _, BI, H, W, D = out_ref.shape

    @pl.when(jnp.logical_and(b == 0, ib == 0))
    def _init():
        for i in range(H):
            eh_ref[i] = Whr_ref[pl.ds(H - 1 - i, H), :]
        for j in range(W):
            ew_ref[j] = Wwr_ref[pl.ds(W - 1 - j, W), :]

    eh = eh_ref[pl.ds(ib * BI, BI)]
    ew = ew_ref[...]
    out_ref[0] = eh[:, :, None, :] + ew[None, :, :, :]


def kernel(x, Wh, Ww):
    B, C, H, W = x.shape
    D = Wh.shape[1]
    BI = 16
    Whr = Wh[::-1]
    Wwr = Ww[::-1]
    return pl.pallas_call(
        _body,
        grid=(B, H // BI),
        in_specs=[
            pl.BlockSpec((2 * H - 1, D), lambda b, ib: (0, 0)),
            pl.BlockSpec((2 * W - 1, D), lambda b, ib: (0, 0)),
        ],
        out_specs=pl.BlockSpec((1, BI, H, W, D), lambda b, ib: (b, ib, 0, 0, 0)),
        out_shape=jax.ShapeDtypeStruct((B, H, H, W, D), jnp.float32),
        scratch_shapes=[
            pltpu.VMEM((H, H, D), jnp.float32),
            pltpu.VMEM((W, W, D), jnp.float32),
        ],
    )(Whr, Wwr)
